# Initial kernel scaffold; baseline (speedup 1.0000x reference)
import jax
import jax.numpy as jnp
from jax.experimental import pallas as pl


def kernel(logits, temperatures, top_ps, min_ps, top_ks, positions, sampling_seeds):
    raise NotImplementedError("write your pallas kernel here")


# speedup vs baseline: 3.6905x; 3.6905x over previous
import functools

import jax
import jax.numpy as jnp
from jax import lax
from jax.experimental import pallas as pl
from jax.experimental.pallas import tpu as pltpu

_BLK = 8
_K = 64
_MASK24 = (1 << 24) - 1
_INV24 = float(1.0 / (1 << 24))
_EPS = 1e-9


def _sampler_kernel(logits_ref, temp_ref, tp_ref, mp_ref, tk_ref, pos_ref,
                    seed_ref, tok_ref, lp_ref, probs_ref, work_ref):
    V = lp_ref.shape[1]
    f32 = jnp.float32
    col = lax.broadcasted_iota(jnp.int32, (_BLK, V), 1)

    x = logits_ref[...] / temp_ref[...]
    mx = jnp.max(x, axis=1, keepdims=True)
    e = jnp.exp(x - mx)
    s = jnp.sum(e, axis=1, keepdims=True)
    probs = e / s
    probs_ref[...] = probs
    lp_ref[...] = jnp.maximum(jnp.log(probs), jnp.finfo(f32).min)
    work_ref[...] = probs

    r64 = lax.broadcasted_iota(jnp.int32, (_BLK, _K), 1)

    def topk_body(i, carry):
        vals, idxs = carry
        w = work_ref[...]
        m = jnp.max(w, axis=1, keepdims=True)
        idx = jnp.max(jnp.where(w == m, col, -1), axis=1, keepdims=True)
        work_ref[...] = jnp.where(col == idx, -1.0, w)
        vals = jnp.where(r64 == i, m, vals)
        idxs = jnp.where(r64 == i, idx, idxs)
        return vals, idxs

    vals, idxs = lax.fori_loop(
        0, _K, topk_body,
        (jnp.zeros((_BLK, _K), f32), jnp.zeros((_BLK, _K), jnp.int32)))

    tri = (lax.broadcasted_iota(jnp.int32, (_K, _K), 0)
           <= lax.broadcasted_iota(jnp.int32, (_K, _K), 1)).astype(f32)
    cs = jnp.dot(vals, tri, preferred_element_type=f32)
    v1 = jnp.where(r64 >= tk_ref[...], 0.0, vals)
    v2 = jnp.where(cs - v1 > tp_ref[...], 0.0, v1)
    thr = v2[:, 0:1] * mp_ref[...]
    v3 = jnp.where(v2 < thr, 0.0, v2)

    ss = seed_ref[...] * 19349663 ^ pos_ref[...] * 73856093
    h64 = ss * 805306457 ^ r64 * 479001599
    u64 = (h64 & _MASK24).astype(f32) * _INV24
    g64 = -jnp.log(-jnp.log(u64 + _EPS) + _EPS)
    pert = jnp.log(v3 + _EPS) + g64
    pmax = jnp.max(pert, axis=1, keepdims=True)
    parg = jnp.min(jnp.where(pert == pmax, r64, _K), axis=1, keepdims=True)
    tok_small = jnp.sum(jnp.where(r64 == parg, idxs, 0), axis=1, keepdims=True)

    hd = ss * 805306457 ^ col * 479001599
    ud = jnp.where(col >= _K, hd & _MASK24, -1)
    dmax = jnp.max(ud, axis=1, keepdims=True)
    rstar = jnp.min(jnp.where(ud == dmax, col, V), axis=1, keepdims=True)
    gd = -jnp.log(-jnp.log(dmax.astype(f32) * _INV24 + _EPS) + _EPS)
    dval = jnp.log(jnp.asarray(_EPS, f32)) + gd
    small_wins = pmax >= dval

    def keys():
        return lax.bitcast_convert_type(probs_ref[...], jnp.int32)

    def bsearch_val(i, carry):
        lo, hi = carry
        mid = lo + (hi - lo) // 2
        cnt = jnp.sum((keys() > mid).astype(jnp.int32), axis=1, keepdims=True)
        pred = cnt <= rstar
        return jnp.where(pred, lo, mid + 1), jnp.where(pred, mid, hi)

    lo0 = jnp.zeros((_BLK, 1), jnp.int32)
    hi0 = jnp.full((_BLK, 1), jnp.int32(2147483647))
    _, vstar = lax.fori_loop(0, 31, bsearch_val, (lo0, hi0))
    cntgt = jnp.sum((keys() > vstar).astype(jnp.int32), axis=1, keepdims=True)
    q = rstar - cntgt

    def bsearch_idx(i, carry):
        lo, hi = carry
        mid = lo + (hi - lo) // 2
        g = jnp.sum(((keys() == vstar) & (col > mid)).astype(jnp.int32),
                    axis=1, keepdims=True)
        pred = g <= q
        return jnp.where(pred, lo, mid + 1), jnp.where(pred, mid, hi)

    hiN = jnp.full((_BLK, 1), jnp.int32(V - 1))
    _, tok_dense = lax.fori_loop(0, 17, bsearch_idx, (lo0, hiN))

    tok_ref[...] = jnp.where(small_wins, tok_small, tok_dense)


def kernel(logits, temperatures, top_ps, min_ps, top_ks, positions,
           sampling_seeds):
    B, V = logits.shape
    f32 = jnp.float32
    i32 = jnp.int32
    row = lambda a, dt: a.reshape(B, 1).astype(dt)
    grid = B // _BLK
    rowspec = pl.BlockSpec((_BLK, 1), lambda i: (i, 0))
    tok, lp = pl.pallas_call(
        _sampler_kernel,
        grid=(grid,),
        in_specs=[pl.BlockSpec((_BLK, V), lambda i: (i, 0))] + [rowspec] * 6,
        out_specs=[rowspec, pl.BlockSpec((_BLK, V), lambda i: (i, 0))],
        out_shape=[jax.ShapeDtypeStruct((B, 1), i32),
                   jax.ShapeDtypeStruct((B, V), f32)],
        scratch_shapes=[pltpu.VMEM((_BLK, V), f32),
                        pltpu.VMEM((_BLK, V), f32)],
    )(logits.astype(f32), row(temperatures, f32), row(top_ps, f32),
      row(min_ps, f32), row(top_ks, i32), row(positions, i32),
      row(sampling_seeds, i32))
    return tok.reshape(-1), lp
